# baseline (speedup 1.0000x reference)
import jax
import jax.numpy as jnp
from jax import lax
from jax.experimental import pallas as pl
from jax.experimental.pallas import tpu as pltpu
from jax.experimental.pallas import tpu_sc as plsc

_VOCAB = 30522
_HIDDEN = 768
_B = 4
_S = 2048
_EPS = 1e-12
_L = 16
_NC, _NS = 2, 16
_NW = _NC * _NS
_PW = _S // _NW
_NSL = _HIDDEN // _L


def _rsqrt16(v):
    i = plsc.bitcast(v, jnp.int32)
    y = plsc.bitcast(jnp.full((_L,), 0x5F3759DF, jnp.int32) - (i >> 1), jnp.float32)
    for _ in range(3):
        y = y * (1.5 - 0.5 * v * y * y)
    return y


def _body(ids_hbm, tt_hbm, word_hbm, pos_hbm, type_hbm, w_hbm, b_hbm, out_hbm,
          idx_v, tt_v, word_v, base_v, type_v, delta_v, w_v, b_v, sem):
    wid = lax.axis_index("s") * _NC + lax.axis_index("c")
    s0 = pl.multiple_of(wid * _PW, _PW)

    pltpu.sync_copy(type_hbm, type_v)
    pltpu.sync_copy(w_hbm, w_v)
    pltpu.sync_copy(b_hbm, b_v)
    pltpu.sync_copy(pos_hbm.at[pl.ds(s0, _PW)], base_v)

    for h in range(_NSL):
        hs = pl.ds(h * _L, _L)
        delta_v[hs] = type_v[1, hs] - type_v[0, hs]

    def _fold_type0(t, carry):
        for h in range(_NSL):
            hs = pl.ds(h * _L, _L)
            base_v[t, hs] = base_v[t, hs] + type_v[0, hs]
        return carry

    lax.fori_loop(0, _PW, _fold_type0, 0)

    inv_h = jnp.float32(1.0 / _HIDDEN)

    def _token_body(t, carry):
        ttf = plsc.load_gather(tt_v, [jnp.full((_L,), t, jnp.int32)])
        ttf = ttf.astype(jnp.float32)
        s = jnp.zeros((_L,), jnp.float32)
        ss = jnp.zeros((_L,), jnp.float32)
        for h in range(_NSL):
            hs = pl.ds(h * _L, _L)
            x = word_v[t, hs] + base_v[t, hs] + ttf * delta_v[hs]
            word_v[t, hs] = x
            s = s + x
            ss = ss + x * x
        mean = jnp.sum(s) * inv_h
        var = jnp.sum(ss) * inv_h - mean * mean
        meanv = jnp.full((_L,), mean)
        r = _rsqrt16(jnp.full((_L,), var + _EPS))
        for h in range(_NSL):
            hs = pl.ds(h * _L, _L)
            word_v[t, hs] = (word_v[t, hs] - meanv) * r * w_v[hs] + b_v[hs]
        return carry

    for b in range(_B):
        base = pl.multiple_of(b * _S + s0, _PW)
        pltpu.sync_copy(ids_hbm.at[pl.ds(base, _PW)], idx_v)
        pltpu.sync_copy(tt_hbm.at[pl.ds(base, _PW)], tt_v)
        pltpu.async_copy(word_hbm.at[idx_v], word_v, sem).wait()
        lax.fori_loop(0, _PW, _token_body, 0)
        pltpu.sync_copy(word_v, out_hbm.at[pl.ds(base, _PW)])


def kernel(input_ids, token_type_ids, word_emb, pos_emb, type_emb, ln_weight, ln_bias):
    ids = input_ids.reshape(-1).astype(jnp.int32)
    tt = token_type_ids.reshape(-1).astype(jnp.int32)
    mesh = plsc.VectorSubcoreMesh(core_axis_name="c", subcore_axis_name="s")
    run = pl.kernel(
        _body,
        out_type=jax.ShapeDtypeStruct((_B * _S, _HIDDEN), jnp.float32),
        mesh=mesh,
        compiler_params=pltpu.CompilerParams(needs_layout_passes=False),
        scratch_types=[
            pltpu.VMEM((_PW,), jnp.int32),
            pltpu.VMEM((_PW,), jnp.int32),
            pltpu.VMEM((_PW, _HIDDEN), jnp.float32),
            pltpu.VMEM((_PW, _HIDDEN), jnp.float32),
            pltpu.VMEM((2, _HIDDEN), jnp.float32),
            pltpu.VMEM((_HIDDEN,), jnp.float32),
            pltpu.VMEM((_HIDDEN,), jnp.float32),
            pltpu.VMEM((_HIDDEN,), jnp.float32),
            pltpu.SemaphoreType.DMA,
        ],
    )
    out = run(ids, tt, word_emb, pos_emb, type_emb, ln_weight, ln_bias)
    return out.reshape(_B, _S, _HIDDEN)
